# baseline (speedup 1.0000x reference)
import functools

import jax
import jax.numpy as jnp
from jax import lax
from jax.experimental import pallas as pl
from jax.experimental.pallas import tpu as pltpu
from jax.experimental.pallas import tpu_sc as plsc

V = 100000
C = 1600000
K = 3
B = 4
ITERS = 4

NC = 2
NS = 16
L = 16

W_PER = C // (NC * NS)
CH0 = 2000
N0 = W_PER // CH0

TILE_C = C // 8
CH = 768
NCH = TILE_C // CH
TAIL = TILE_C - NCH * CH
ROWS = K * CH // 128
TROWS = (K * TAIL + 64) // 128
DPAD = 192
DSZ = 2 * V + DPAD
ZSL = DSZ // NS
PIECE = K * CH
NPIECE = V // PIECE
PTAIL = V - NPIECE * PIECE

_mesh = plsc.VectorSubcoreMesh(core_axis_name="c", subcore_axis_name="s")


def _ds8(off, n):
    return pl.ds(pl.multiple_of(off, 8), n)


@functools.partial(
    pl.kernel,
    out_type=(jax.ShapeDtypeStruct((K * C,), jnp.int32),
              jax.ShapeDtypeStruct((K * C,), jnp.float32)),
    mesh=_mesh,
    compiler_params=pltpu.CompilerParams(needs_layout_passes=False),
    scratch_types=[
        pltpu.VMEM((V,), jnp.float32),
        pltpu.VMEM((K * CH0,), jnp.int32),
        pltpu.VMEM((K * CH0,), jnp.int32),
        pltpu.VMEM((K * CH0,), jnp.float32),
    ],
)
def _phase0(claus_hbm, vw_hbm, lit_hbm, sw_hbm, wtab, rawb, litb, swb):
    c = lax.axis_index("c")
    s = lax.axis_index("s")
    wid = s * NC + c
    base = wid * W_PER
    pltpu.sync_copy(vw_hbm, wtab)
    iota3 = lax.iota(jnp.int32, L) * K

    @pl.loop(0, N0)
    def _chunk(ch):
        cb = base + ch * CH0
        pltpu.sync_copy(claus_hbm.at[_ds8(K * cb, K * CH0)], rawb)

        @pl.loop(0, CH0 // L)
        def _vec(i):
            for k in range(K):
                lit = plsc.load_gather(rawb, [i * (K * L) + iota3 + k])
                var = lax.shift_right_logical(lit, 1)
                w = plsc.load_gather(wtab, [var])
                sgn = jnp.where(lax.rem(lit, 2) == 0, 0.1, -0.1)
                litb[pl.ds(k * CH0 + i * L, L)] = lit
                swb[pl.ds(k * CH0 + i * L, L)] = sgn.astype(jnp.float32) * w

        for k in range(K):
            pltpu.sync_copy(litb.at[pl.ds(k * CH0, CH0)],
                            lit_hbm.at[_ds8(k * C + cb, CH0)])
            pltpu.sync_copy(swb.at[pl.ds(k * CH0, CH0)],
                            sw_hbm.at[_ds8(k * C + cb, CH0)])


@functools.partial(
    pl.kernel,
    out_type=(jax.ShapeDtypeStruct((B * V,), jnp.float32),
              jax.ShapeDtypeStruct((B, L), jnp.float32)),
    mesh=_mesh,
    compiler_params=pltpu.CompilerParams(needs_layout_passes=False),
    scratch_types=[
        pltpu.VMEM((V,), jnp.float32),
        pltpu.VMEM((K * CH,), jnp.int32),
        pltpu.VMEM((K * CH,), jnp.int32),
        pltpu.VMEM((K * CH,), jnp.float32),
        pltpu.VMEM((K * CH,), jnp.float32),
        pltpu.VMEM((ROWS, 128), jnp.int32),
        pltpu.VMEM((ROWS, 128), jnp.float32),
        pltpu.VMEM((L,), jnp.float32),
        pltpu.VMEM((8, L), jnp.float32),
        pltpu.VMEM_SHARED((DSZ,), jnp.float32),
        pltpu.VMEM_SHARED((NS, L), jnp.float32),
        pltpu.SemaphoreType.DMA,
        pltpu.SemaphoreType.DMA,
        pltpu.SemaphoreType.DMA,
    ],
)
def _main(lit_hbm, sw_hbm, a0_hbm, a_out, sat_out,
          a_loc, litb0, litb1, swb0, swb1, idxb, cb, tb, pb,
          delta, parts, sem_in0, sem_in1, sem_sc):
    c = lax.axis_index("c")
    s = lax.axis_index("s")
    b_in = s // 8
    p = lax.rem(s, 8)
    b = 2 * c + b_in
    boff = b_in * V
    tbase = p * TILE_C

    litb = (litb0, litb1)
    swb = (swb0, swb1)
    sem_in = (sem_in0, sem_in1)

    pltpu.sync_copy(a0_hbm.at[_ds8(b * V, V)], a_loc)

    zero16 = jnp.zeros((L,), jnp.float32)
    ones = jnp.ones((L,), jnp.float32)
    iota16 = lax.iota(jnp.int32, L)

    def in_copies(ch, par, n, with_sw):
        cbs = tbase + ch * CH
        cps = []
        for k in range(K):
            cps.append((lit_hbm.at[_ds8(k * C + cbs, n)],
                        litb[par].at[pl.ds(k * n, n)]))
            if with_sw:
                cps.append((sw_hbm.at[_ds8(k * C + cbs, n)],
                            swb[par].at[pl.ds(k * n, n)]))
        return cps

    def start_in(ch, par, n, with_sw):
        for src_, dst in in_copies(ch, par, n, with_sw):
            pltpu.async_copy(src_, dst, sem_in[par])

    def wait_in(ch, par, n, with_sw):
        for src_, dst in in_copies(ch, par, n, with_sw):
            pltpu.make_async_copy(src_, dst, sem_in[par]).wait()

    def fire_scatter(rows):
        for j in range(rows):
            pltpu.async_copy(cb.at[j], delta.at[idxb.at[j]], sem_sc,
                             add=True)

    def drain_scatter(rows):
        for j in range(rows):
            pltpu.make_async_copy(cb.at[j], delta.at[idxb.at[j]],
                                  sem_sc).wait()

    def gather_litvals(par, i, n):
        vars_ = []
        litvals = []
        for k in range(K):
            lit = litb[par][pl.ds(k * n + i * L, L)]
            var = lax.shift_right_logical(lit, 1)
            g = plsc.load_gather(a_loc, [var])
            lv = jnp.where(lax.rem(lit, 2) == 0, g, 1.0 - g)
            vars_.append(var)
            litvals.append(lv)
        return vars_, litvals

    def body(ch, par, n, rows, drain, prefetch):
        wait_in(ch, par, n, True)
        if drain == "always":
            drain_scatter(ROWS)
        elif drain == "cond":
            @pl.when(ch >= 1)
            def _():
                drain_scatter(ROWS)

        @pl.loop(0, n // L)
        def _vec(i):
            vars_, litvals = gather_litvals(par, i, n)
            viol = 1.0 - jnp.maximum(jnp.maximum(litvals[0], litvals[1]),
                                     litvals[2])
            for k in range(K):
                swv = swb[par][pl.ds(k * n + i * L, L)]
                off = k * n + i * L
                r = lax.shift_right_logical(off, 7)
                col = lax.rem(off, 128)
                cb[r, pl.ds(col, L)] = viol * swv
                idxb[r, pl.ds(col, L)] = vars_[k] + boff

        if n == TAIL:
            for j in range(4):
                off = K * TAIL + j * L
                cb[off >> 7, pl.ds(off % 128, L)] = zero16
                idxb[off >> 7, pl.ds(off % 128, L)] = (
                    2 * V + j * L + iota16)
        fire_scatter(rows)
        if prefetch:
            @pl.when(ch + 2 < NCH)
            def _():
                start_in(ch + 2, par, CH, True)

            @pl.when(ch + 2 == NCH)
            def _():
                start_in(NCH, par, TAIL, True)

    @pl.loop(0, ITERS)
    def _iter(it):
        @pl.loop(0, PIECE // L)
        def _z(i):
            swb0[pl.ds(i * L, L)] = zero16

        zb = s * ZSL
        for t in range(5):
            pltpu.sync_copy(swb0.at[pl.ds(0, PIECE)],
                            delta.at[_ds8(zb + t * PIECE, PIECE)])
        pltpu.sync_copy(swb0.at[pl.ds(0, ZSL - 5 * PIECE)],
                        delta.at[_ds8(zb + 5 * PIECE, ZSL - 5 * PIECE)])
        plsc.subcore_barrier()

        with jax.named_scope("scat_pass"):
            start_in(0, 0, CH, True)
            start_in(1, 1, CH, True)

            @pl.loop(0, NCH, step=2)
            def _pair(ch):
                body(ch, 0, CH, ROWS, "cond", True)
                body(ch + 1, 1, CH, ROWS, "always", True)

            body(NCH, 0, TAIL, TROWS, "always", False)
            drain_scatter(TROWS)

        plsc.subcore_barrier()

        def apply_piece(jbase, n):
            pltpu.sync_copy(delta.at[_ds8(boff + jbase, n)],
                            swb0.at[pl.ds(0, n)])

            @pl.loop(0, n // L)
            def _v(i):
                av = a_loc[pl.ds(jbase + i * L, L)]
                dv = swb0[pl.ds(i * L, L)]
                a_loc[pl.ds(jbase + i * L, L)] = jnp.clip(av + dv, 0.0, 1.0)

        with jax.named_scope("upd_pass"):
            @pl.loop(0, NPIECE)
            def _u(j):
                apply_piece(j * PIECE, PIECE)

            apply_piece(NPIECE * PIECE, PTAIL)
        plsc.subcore_barrier()

    @pl.when(lax.rem(p, 2) == 0)
    def _():
        pltpu.sync_copy(a_loc.at[_ds8(p * (V // 8), 2 * (V // 8))],
                        a_out.at[_ds8(b * V + p * (V // 8), 2 * (V // 8))])

    def sat_body(ch, par, n, pv, prefetch):
        wait_in(ch, par, n, False)

        @pl.loop(0, n // L, init_carry=pv)
        def _vec(i, acc):
            _, litvals = gather_litvals(par, i, n)
            csat = jnp.maximum(jnp.maximum(litvals[0], litvals[1]),
                               litvals[2])
            return acc * csat

        if prefetch:
            @pl.when(ch + 2 < NCH)
            def _():
                start_in(ch + 2, par, CH, False)

            @pl.when(ch + 2 == NCH)
            def _():
                start_in(NCH, par, TAIL, False)
        return _vec

    with jax.named_scope("sat_pass"):
        start_in(0, 0, CH, False)
        start_in(1, 1, CH, False)

        @pl.loop(0, NCH, step=2, init_carry=ones)
        def _sat(ch, pv):
            pv = sat_body(ch, 0, CH, pv, True)
            return sat_body(ch + 1, 1, CH, pv, True)

        pv = sat_body(NCH, 0, TAIL, _sat, False)

    for sh in (8, 4, 2, 1):
        tb[...] = pv
        rolled = plsc.load_gather(tb, [lax.rem(iota16 + sh, L)])
        pv = pv * rolled
    tb[...] = pv
    pltpu.sync_copy(tb, parts.at[s])
    plsc.subcore_barrier()

    @pl.when(p == 0)
    def _():
        pltpu.sync_copy(parts.at[pl.ds(b_in * 8, 8)], pb)
        sv = ones
        for k in range(8):
            sv = sv * pb[k]
        tb[...] = sv
        pltpu.sync_copy(tb, sat_out.at[b])


def kernel(clauses, initial_assignment, variable_weights, clause_weights):
    del clause_weights
    a0 = initial_assignment.reshape(-1)
    lit_t, sw = _phase0(clauses.reshape(-1), variable_weights)
    a_flat, sat = _main(lit_t, sw, a0)
    return a_flat.reshape(B, V), sat[:, 0]


# speedup vs baseline: 2.6564x; 2.6564x over previous
import functools

import jax
import jax.numpy as jnp
from jax import lax
from jax.experimental import pallas as pl
from jax.experimental.pallas import tpu as pltpu
from jax.experimental.pallas import tpu_sc as plsc

V = 100000
C = 1600000
K = 3
B = 4
ITERS = 4

NC = 2
NS = 16
L = 16

W_PER = C // (NC * NS)
CH0 = 2000
N0 = W_PER // CH0

TILE_C = C // 8
CH = 768
NCH = TILE_C // CH
TAIL = TILE_C - NCH * CH
ROWS = K * CH // 128
TROWS = (K * TAIL + 64) // 128
DPAD = 192
DSZ = 2 * V + DPAD
ZSL = DSZ // NS
PIECE = K * CH
NPIECE = V // PIECE
PTAIL = V - NPIECE * PIECE

_mesh = plsc.VectorSubcoreMesh(core_axis_name="c", subcore_axis_name="s")


def _ds8(off, n):
    return pl.ds(pl.multiple_of(off, 8), n)


@functools.partial(
    pl.kernel,
    out_type=jax.ShapeDtypeStruct((K * C,), jnp.float32),
    mesh=_mesh,
    compiler_params=pltpu.CompilerParams(needs_layout_passes=False),
    scratch_types=[
        pltpu.VMEM((V,), jnp.float32),
        pltpu.VMEM((K * CH0,), jnp.int32),
        pltpu.VMEM((K * CH0,), jnp.float32),
    ],
)
def _phase0(lit_hbm, vw_hbm, sw_hbm, wtab, litb, swb):
    c = lax.axis_index("c")
    s = lax.axis_index("s")
    wid = s * NC + c
    base = wid * W_PER
    pltpu.sync_copy(vw_hbm, wtab)

    @pl.loop(0, N0)
    def _chunk(ch):
        cb = base + ch * CH0
        for k in range(K):
            pltpu.sync_copy(lit_hbm.at[_ds8(k * C + cb, CH0)],
                            litb.at[pl.ds(k * CH0, CH0)])

        @pl.loop(0, CH0 // L)
        def _vec(i):
            for k in range(K):
                lit = litb[pl.ds(k * CH0 + i * L, L)]
                var = lax.shift_right_logical(lit, 1)
                w = plsc.load_gather(wtab, [var])
                sgn = jnp.where(lax.rem(lit, 2) == 0, 0.1, -0.1)
                swb[pl.ds(k * CH0 + i * L, L)] = sgn.astype(jnp.float32) * w

        for k in range(K):
            pltpu.sync_copy(swb.at[pl.ds(k * CH0, CH0)],
                            sw_hbm.at[_ds8(k * C + cb, CH0)])


@functools.partial(
    pl.kernel,
    out_type=(jax.ShapeDtypeStruct((B * V,), jnp.float32),
              jax.ShapeDtypeStruct((B, L), jnp.float32)),
    mesh=_mesh,
    compiler_params=pltpu.CompilerParams(needs_layout_passes=False),
    scratch_types=[
        pltpu.VMEM((V,), jnp.float32),
        pltpu.VMEM((K * CH,), jnp.int32),
        pltpu.VMEM((K * CH,), jnp.int32),
        pltpu.VMEM((K * CH,), jnp.float32),
        pltpu.VMEM((K * CH,), jnp.float32),
        pltpu.VMEM((ROWS, 128), jnp.int32),
        pltpu.VMEM((ROWS, 128), jnp.float32),
        pltpu.VMEM((L,), jnp.float32),
        pltpu.VMEM((8, L), jnp.float32),
        pltpu.VMEM_SHARED((DSZ,), jnp.float32),
        pltpu.VMEM_SHARED((NS, L), jnp.float32),
        pltpu.SemaphoreType.DMA,
        pltpu.SemaphoreType.DMA,
        pltpu.SemaphoreType.DMA,
    ],
)
def _main(lit_hbm, sw_hbm, a0_hbm, a_out, sat_out,
          a_loc, litb0, litb1, swb0, swb1, idxb, cb, tb, pb,
          delta, parts, sem_in0, sem_in1, sem_sc):
    c = lax.axis_index("c")
    s = lax.axis_index("s")
    b_in = s // 8
    p = lax.rem(s, 8)
    b = 2 * c + b_in
    boff = b_in * V
    tbase = p * TILE_C

    litb = (litb0, litb1)
    swb = (swb0, swb1)
    sem_in = (sem_in0, sem_in1)

    pltpu.sync_copy(a0_hbm.at[_ds8(b * V, V)], a_loc)

    zero16 = jnp.zeros((L,), jnp.float32)
    ones = jnp.ones((L,), jnp.float32)
    iota16 = lax.iota(jnp.int32, L)

    def in_copies(ch, par, n, with_sw):
        cbs = tbase + ch * CH
        cps = []
        for k in range(K):
            cps.append((lit_hbm.at[_ds8(k * C + cbs, n)],
                        litb[par].at[pl.ds(k * n, n)]))
            if with_sw:
                cps.append((sw_hbm.at[_ds8(k * C + cbs, n)],
                            swb[par].at[pl.ds(k * n, n)]))
        return cps

    def start_in(ch, par, n, with_sw):
        for src_, dst in in_copies(ch, par, n, with_sw):
            pltpu.async_copy(src_, dst, sem_in[par])

    def wait_in(ch, par, n, with_sw):
        for src_, dst in in_copies(ch, par, n, with_sw):
            pltpu.make_async_copy(src_, dst, sem_in[par]).wait()

    def fire_scatter(rows):
        for j in range(rows):
            pltpu.async_copy(cb.at[j], delta.at[idxb.at[j]], sem_sc,
                             add=True)

    def drain_scatter(rows):
        for j in range(rows):
            pltpu.make_async_copy(cb.at[j], delta.at[idxb.at[j]],
                                  sem_sc).wait()

    def gather_litvals(par, i, n):
        vars_ = []
        litvals = []
        for k in range(K):
            lit = litb[par][pl.ds(k * n + i * L, L)]
            var = lax.shift_right_logical(lit, 1)
            g = plsc.load_gather(a_loc, [var])
            lv = jnp.where(lax.rem(lit, 2) == 0, g, 1.0 - g)
            vars_.append(var)
            litvals.append(lv)
        return vars_, litvals

    def body(ch, par, n, rows, drain, prefetch):
        wait_in(ch, par, n, True)
        if drain == "always":
            drain_scatter(ROWS)
        elif drain == "cond":
            @pl.when(ch >= 1)
            def _():
                drain_scatter(ROWS)

        @pl.loop(0, n // L)
        def _vec(i):
            vars_, litvals = gather_litvals(par, i, n)
            viol = 1.0 - jnp.maximum(jnp.maximum(litvals[0], litvals[1]),
                                     litvals[2])
            for k in range(K):
                swv = swb[par][pl.ds(k * n + i * L, L)]
                off = k * n + i * L
                r = lax.shift_right_logical(off, 7)
                col = lax.rem(off, 128)
                cb[r, pl.ds(col, L)] = viol * swv
                idxb[r, pl.ds(col, L)] = vars_[k] + boff

        if n == TAIL:
            for j in range(4):
                off = K * TAIL + j * L
                cb[off >> 7, pl.ds(off % 128, L)] = zero16
                idxb[off >> 7, pl.ds(off % 128, L)] = (
                    2 * V + j * L + iota16)
        fire_scatter(rows)
        if prefetch:
            @pl.when(ch + 2 < NCH)
            def _():
                start_in(ch + 2, par, CH, True)

            @pl.when(ch + 2 == NCH)
            def _():
                start_in(NCH, par, TAIL, True)

    @pl.loop(0, ITERS)
    def _iter(it):
        @pl.loop(0, PIECE // L)
        def _z(i):
            swb0[pl.ds(i * L, L)] = zero16

        zb = s * ZSL
        for t in range(5):
            pltpu.sync_copy(swb0.at[pl.ds(0, PIECE)],
                            delta.at[_ds8(zb + t * PIECE, PIECE)])
        pltpu.sync_copy(swb0.at[pl.ds(0, ZSL - 5 * PIECE)],
                        delta.at[_ds8(zb + 5 * PIECE, ZSL - 5 * PIECE)])
        plsc.subcore_barrier()

        with jax.named_scope("scat_pass"):
            start_in(0, 0, CH, True)
            start_in(1, 1, CH, True)

            @pl.loop(0, NCH, step=2)
            def _pair(ch):
                body(ch, 0, CH, ROWS, "cond", True)
                body(ch + 1, 1, CH, ROWS, "always", True)

            body(NCH, 0, TAIL, TROWS, "always", False)
            drain_scatter(TROWS)

        plsc.subcore_barrier()

        def apply_piece(jbase, n):
            pltpu.sync_copy(delta.at[_ds8(boff + jbase, n)],
                            swb0.at[pl.ds(0, n)])

            @pl.loop(0, n // L)
            def _v(i):
                av = a_loc[pl.ds(jbase + i * L, L)]
                dv = swb0[pl.ds(i * L, L)]
                a_loc[pl.ds(jbase + i * L, L)] = jnp.clip(av + dv, 0.0, 1.0)

        with jax.named_scope("upd_pass"):
            @pl.loop(0, NPIECE)
            def _u(j):
                apply_piece(j * PIECE, PIECE)

            apply_piece(NPIECE * PIECE, PTAIL)
        plsc.subcore_barrier()

    @pl.when(lax.rem(p, 2) == 0)
    def _():
        pltpu.sync_copy(a_loc.at[_ds8(p * (V // 8), 2 * (V // 8))],
                        a_out.at[_ds8(b * V + p * (V // 8), 2 * (V // 8))])

    def sat_body(ch, par, n, pv, prefetch):
        wait_in(ch, par, n, False)

        @pl.loop(0, n // L, init_carry=pv)
        def _vec(i, acc):
            _, litvals = gather_litvals(par, i, n)
            csat = jnp.maximum(jnp.maximum(litvals[0], litvals[1]),
                               litvals[2])
            return acc * csat

        if prefetch:
            @pl.when(ch + 2 < NCH)
            def _():
                start_in(ch + 2, par, CH, False)

            @pl.when(ch + 2 == NCH)
            def _():
                start_in(NCH, par, TAIL, False)
        return _vec

    with jax.named_scope("sat_pass"):
        start_in(0, 0, CH, False)
        start_in(1, 1, CH, False)

        @pl.loop(0, NCH, step=2, init_carry=ones)
        def _sat(ch, pv):
            pv = sat_body(ch, 0, CH, pv, True)
            return sat_body(ch + 1, 1, CH, pv, True)

        pv = sat_body(NCH, 0, TAIL, _sat, False)

    for sh in (8, 4, 2, 1):
        tb[...] = pv
        rolled = plsc.load_gather(tb, [lax.rem(iota16 + sh, L)])
        pv = pv * rolled
    tb[...] = pv
    pltpu.sync_copy(tb, parts.at[s])
    plsc.subcore_barrier()

    @pl.when(p == 0)
    def _():
        pltpu.sync_copy(parts.at[pl.ds(b_in * 8, 8)], pb)
        sv = ones
        for k in range(8):
            sv = sv * pb[k]
        tb[...] = sv
        pltpu.sync_copy(tb, sat_out.at[b])


def kernel(clauses, initial_assignment, variable_weights, clause_weights):
    del clause_weights
    lit_t = clauses.T.reshape(-1)
    a0 = initial_assignment.reshape(-1)
    sw = _phase0(lit_t, variable_weights)
    a_flat, sat = _main(lit_t, sw, a0)
    return a_flat.reshape(B, V), sat[:, 0]


# speedup vs baseline: 2.6838x; 1.0103x over previous
import functools

import jax
import jax.numpy as jnp
from jax import lax
from jax.experimental import pallas as pl
from jax.experimental.pallas import tpu as pltpu
from jax.experimental.pallas import tpu_sc as plsc

V = 100000
C = 1600000
K = 3
B = 4
ITERS = 4

NC = 2
NS = 16
L = 16

W_PER = C // (NC * NS)
CH0 = 2000
N0 = W_PER // CH0

TILE_C = C // 8
CH = 768
NCH = TILE_C // CH
TAIL = TILE_C - NCH * CH
ROWS = K * CH // 128
TROWS = (K * TAIL + 64) // 128
DPAD = 192
DSZ = 2 * V + DPAD
ZSL = DSZ // NS
PIECE = K * CH
NPIECE = V // PIECE
PTAIL = V - NPIECE * PIECE

_mesh = plsc.VectorSubcoreMesh(core_axis_name="c", subcore_axis_name="s")


def _ds8(off, n):
    return pl.ds(pl.multiple_of(off, 8), n)


@functools.partial(
    pl.kernel,
    out_type=jax.ShapeDtypeStruct((K * C,), jnp.float32),
    mesh=_mesh,
    compiler_params=pltpu.CompilerParams(needs_layout_passes=False),
    scratch_types=[
        pltpu.VMEM((V,), jnp.float32),
        pltpu.VMEM((K * CH0,), jnp.int32),
        pltpu.VMEM((K * CH0,), jnp.float32),
    ],
)
def _phase0(lit_hbm, vw_hbm, sw_hbm, wtab, litb, swb):
    c = lax.axis_index("c")
    s = lax.axis_index("s")
    wid = s * NC + c
    base = wid * W_PER
    pltpu.sync_copy(vw_hbm, wtab)

    @pl.loop(0, N0)
    def _chunk(ch):
        cb = base + ch * CH0
        for k in range(K):
            pltpu.sync_copy(lit_hbm.at[_ds8(k * C + cb, CH0)],
                            litb.at[pl.ds(k * CH0, CH0)])

        @pl.loop(0, CH0 // L)
        def _vec(i):
            for k in range(K):
                lit = litb[pl.ds(k * CH0 + i * L, L)]
                var = lax.shift_right_logical(lit, 1)
                w = plsc.load_gather(wtab, [var])
                sgn = jnp.where(lax.rem(lit, 2) == 0, 0.1, -0.1)
                swb[pl.ds(k * CH0 + i * L, L)] = sgn.astype(jnp.float32) * w

        for k in range(K):
            pltpu.sync_copy(swb.at[pl.ds(k * CH0, CH0)],
                            sw_hbm.at[_ds8(k * C + cb, CH0)])


@functools.partial(
    pl.kernel,
    out_type=(jax.ShapeDtypeStruct((B * V,), jnp.float32),
              jax.ShapeDtypeStruct((B, L), jnp.float32)),
    mesh=_mesh,
    compiler_params=pltpu.CompilerParams(needs_layout_passes=False),
    scratch_types=[
        pltpu.VMEM((V,), jnp.float32),
        pltpu.VMEM((K * CH,), jnp.int32),
        pltpu.VMEM((K * CH,), jnp.int32),
        pltpu.VMEM((K * CH,), jnp.float32),
        pltpu.VMEM((K * CH,), jnp.float32),
        pltpu.VMEM((ROWS, 128), jnp.int32),
        pltpu.VMEM((ROWS, 128), jnp.float32),
        pltpu.VMEM((L,), jnp.float32),
        pltpu.VMEM((8, L), jnp.float32),
        pltpu.VMEM_SHARED((DSZ,), jnp.float32),
        pltpu.VMEM_SHARED((NS, L), jnp.float32),
        pltpu.SemaphoreType.DMA,
        pltpu.SemaphoreType.DMA,
        pltpu.SemaphoreType.DMA,
    ],
)
def _main(lit_hbm, sw_hbm, a0_hbm, a_out, sat_out,
          a_loc, litb0, litb1, swb0, swb1, idxb, cb, tb, pb,
          delta, parts, sem_in0, sem_in1, sem_sc):
    c = lax.axis_index("c")
    s = lax.axis_index("s")
    b_in = s // 8
    p = lax.rem(s, 8)
    b = 2 * c + b_in
    boff = b_in * V
    tbase = p * TILE_C

    litb = (litb0, litb1)
    swb = (swb0, swb1)
    sem_in = (sem_in0, sem_in1)

    pltpu.sync_copy(a0_hbm.at[_ds8(b * V, V)], a_loc)

    zero16 = jnp.zeros((L,), jnp.float32)
    ones = jnp.ones((L,), jnp.float32)
    iota16 = lax.iota(jnp.int32, L)

    def in_copies(ch, par, n, with_sw):
        cbs = tbase + ch * CH
        cps = []
        for k in range(K):
            cps.append((lit_hbm.at[_ds8(k * C + cbs, n)],
                        litb[par].at[pl.ds(k * n, n)]))
            if with_sw:
                cps.append((sw_hbm.at[_ds8(k * C + cbs, n)],
                            swb[par].at[pl.ds(k * n, n)]))
        return cps

    def start_in(ch, par, n, with_sw):
        for src_, dst in in_copies(ch, par, n, with_sw):
            pltpu.async_copy(src_, dst, sem_in[par])

    def wait_in(ch, par, n, with_sw):
        for src_, dst in in_copies(ch, par, n, with_sw):
            pltpu.make_async_copy(src_, dst, sem_in[par]).wait()

    def fire_scatter(rows):
        for j in range(rows):
            pltpu.async_copy(cb.at[j], delta.at[idxb.at[j]], sem_sc,
                             add=True)

    def drain_scatter(rows):
        for j in range(rows):
            pltpu.make_async_copy(cb.at[j], delta.at[idxb.at[j]],
                                  sem_sc).wait()

    def gather_litvals(par, i, n):
        vars_ = []
        litvals = []
        for k in range(K):
            lit = litb[par][pl.ds(k * n + i * L, L)]
            var = lax.shift_right_logical(lit, 1)
            g = plsc.load_gather(a_loc, [var])
            lv = jnp.where(lax.rem(lit, 2) == 0, g, 1.0 - g)
            vars_.append(var)
            litvals.append(lv)
        return vars_, litvals

    def body(ch, par, n, rows, drain, prefetch):
        wait_in(ch, par, n, True)
        if drain == "always":
            drain_scatter(ROWS)
        elif drain == "cond":
            @pl.when(ch >= 1)
            def _():
                drain_scatter(ROWS)

        @pl.loop(0, n // L)
        def _vec(i):
            vars_, litvals = gather_litvals(par, i, n)
            viol = 1.0 - jnp.maximum(jnp.maximum(litvals[0], litvals[1]),
                                     litvals[2])
            for k in range(K):
                swv = swb[par][pl.ds(k * n + i * L, L)]
                off = k * n + i * L
                r = lax.shift_right_logical(off, 7)
                col = lax.rem(off, 128)
                cb[r, pl.ds(col, L)] = viol * swv
                idxb[r, pl.ds(col, L)] = vars_[k] + boff

        if n == TAIL:
            for j in range(4):
                off = K * TAIL + j * L
                cb[off >> 7, pl.ds(off % 128, L)] = zero16
                idxb[off >> 7, pl.ds(off % 128, L)] = (
                    2 * V + j * L + iota16)
        fire_scatter(rows)
        if prefetch:
            @pl.when(ch + 2 < NCH)
            def _():
                start_in(ch + 2, par, CH, True)

            @pl.when(ch + 2 == NCH)
            def _():
                start_in(NCH, par, TAIL, True)

    @pl.loop(0, ITERS)
    def _iter(it):
        @pl.loop(0, PIECE // L)
        def _z(i):
            swb0[pl.ds(i * L, L)] = zero16

        zb = s * ZSL
        for t in range(5):
            pltpu.sync_copy(swb0.at[pl.ds(0, PIECE)],
                            delta.at[_ds8(zb + t * PIECE, PIECE)])
        pltpu.sync_copy(swb0.at[pl.ds(0, ZSL - 5 * PIECE)],
                        delta.at[_ds8(zb + 5 * PIECE, ZSL - 5 * PIECE)])
        plsc.subcore_barrier()

        with jax.named_scope("scat_pass"):
            start_in(0, 0, CH, True)
            start_in(1, 1, CH, True)

            @pl.loop(0, NCH, step=2)
            def _pair(ch):
                body(ch, 0, CH, ROWS, "cond", True)
                body(ch + 1, 1, CH, ROWS, "always", True)

            body(NCH, 0, TAIL, TROWS, "always", False)
            drain_scatter(TROWS)

        plsc.subcore_barrier()

        def start_piece(j, par, n):
            pltpu.async_copy(delta.at[_ds8(boff + j * PIECE, n)],
                             swb[par].at[pl.ds(0, n)], sem_in[par])

        def apply_piece(j, par, n, pre_j, pre_n):
            pltpu.make_async_copy(delta.at[_ds8(boff + j * PIECE, n)],
                                  swb[par].at[pl.ds(0, n)],
                                  sem_in[par]).wait()
            if pre_n:
                start_piece(pre_j, par, pre_n)
            jb = j * PIECE

            @pl.loop(0, n // L)
            def _v(i):
                av = a_loc[pl.ds(jb + i * L, L)]
                dv = swb[par][pl.ds(i * L, L)]
                a_loc[pl.ds(jb + i * L, L)] = jnp.clip(av + dv, 0.0, 1.0)

        with jax.named_scope("upd_pass"):
            start_piece(0, 0, PIECE)
            start_piece(1, 1, PIECE)

            @pl.loop(0, NPIECE - 3, step=2)
            def _u(j):
                apply_piece(j, 0, PIECE, j + 2, PIECE)
                apply_piece(j + 1, 1, PIECE, j + 3, PIECE)

            apply_piece(NPIECE - 3, 0, PIECE, NPIECE - 1, PIECE)
            apply_piece(NPIECE - 2, 1, PIECE, NPIECE, PTAIL)
            apply_piece(NPIECE - 1, 0, PIECE, 0, 0)
            apply_piece(NPIECE, 1, PTAIL, 0, 0)
        plsc.subcore_barrier()

    @pl.when(lax.rem(p, 2) == 0)
    def _():
        pltpu.sync_copy(a_loc.at[_ds8(p * (V // 8), 2 * (V // 8))],
                        a_out.at[_ds8(b * V + p * (V // 8), 2 * (V // 8))])

    def sat_body(ch, par, n, pv, prefetch):
        wait_in(ch, par, n, False)

        @pl.loop(0, n // L, init_carry=pv)
        def _vec(i, acc):
            _, litvals = gather_litvals(par, i, n)
            csat = jnp.maximum(jnp.maximum(litvals[0], litvals[1]),
                               litvals[2])
            return acc * csat

        if prefetch:
            @pl.when(ch + 2 < NCH)
            def _():
                start_in(ch + 2, par, CH, False)

            @pl.when(ch + 2 == NCH)
            def _():
                start_in(NCH, par, TAIL, False)
        return _vec

    with jax.named_scope("sat_pass"):
        start_in(0, 0, CH, False)
        start_in(1, 1, CH, False)

        @pl.loop(0, NCH, step=2, init_carry=ones)
        def _sat(ch, pv):
            pv = sat_body(ch, 0, CH, pv, True)
            return sat_body(ch + 1, 1, CH, pv, True)

        pv = sat_body(NCH, 0, TAIL, _sat, False)

    for sh in (8, 4, 2, 1):
        tb[...] = pv
        rolled = plsc.load_gather(tb, [lax.rem(iota16 + sh, L)])
        pv = pv * rolled
    tb[...] = pv
    pltpu.sync_copy(tb, parts.at[s])
    plsc.subcore_barrier()

    @pl.when(p == 0)
    def _():
        pltpu.sync_copy(parts.at[pl.ds(b_in * 8, 8)], pb)
        sv = ones
        for k in range(8):
            sv = sv * pb[k]
        tb[...] = sv
        pltpu.sync_copy(tb, sat_out.at[b])


def kernel(clauses, initial_assignment, variable_weights, clause_weights):
    del clause_weights
    lit_t = clauses.T.reshape(-1)
    a0 = initial_assignment.reshape(-1)
    sw = _phase0(lit_t, variable_weights)
    a_flat, sat = _main(lit_t, sw, a0)
    return a_flat.reshape(B, V), sat[:, 0]


# speedup vs baseline: 2.6843x; 1.0002x over previous
import functools

import jax
import jax.numpy as jnp
from jax import lax
from jax.experimental import pallas as pl
from jax.experimental.pallas import tpu as pltpu
from jax.experimental.pallas import tpu_sc as plsc

V = 100000
C = 1600000
K = 3
B = 4
ITERS = 4

NC = 2
NS = 16
L = 16

W_PER = C // (NC * NS)
CH0 = 2000
N0 = W_PER // CH0

TILE_C = C // 8
CH = 768
NCH = TILE_C // CH
TAIL = TILE_C - NCH * CH
ROWS = K * CH // 128
TROWS = (K * TAIL + 64) // 128
DPAD = 192
DSZ = 2 * V + DPAD
ZSL = DSZ // NS
PIECE = K * CH
NPIECE = V // PIECE
PTAIL = V - NPIECE * PIECE

_mesh = plsc.VectorSubcoreMesh(core_axis_name="c", subcore_axis_name="s")


def _ds8(off, n):
    return pl.ds(pl.multiple_of(off, 8), n)


@functools.partial(
    pl.kernel,
    out_type=jax.ShapeDtypeStruct((K * C,), jnp.float32),
    mesh=_mesh,
    compiler_params=pltpu.CompilerParams(needs_layout_passes=False),
    scratch_types=[
        pltpu.VMEM((V,), jnp.float32),
        pltpu.VMEM((K * CH0,), jnp.int32),
        pltpu.VMEM((K * CH0,), jnp.float32),
    ],
)
def _phase0(lit_hbm, vw_hbm, sw_hbm, wtab, litb, swb):
    c = lax.axis_index("c")
    s = lax.axis_index("s")
    wid = s * NC + c
    base = wid * W_PER
    pltpu.sync_copy(vw_hbm, wtab)

    @pl.loop(0, N0)
    def _chunk(ch):
        cb = base + ch * CH0
        for k in range(K):
            pltpu.sync_copy(lit_hbm.at[_ds8(k * C + cb, CH0)],
                            litb.at[pl.ds(k * CH0, CH0)])

        @pl.loop(0, CH0 // L)
        def _vec(i):
            for k in range(K):
                lit = litb[pl.ds(k * CH0 + i * L, L)]
                var = lax.shift_right_logical(lit, 1)
                w = plsc.load_gather(wtab, [var])
                sgn = jnp.where(lax.rem(lit, 2) == 0, 0.1, -0.1)
                swb[pl.ds(k * CH0 + i * L, L)] = sgn.astype(jnp.float32) * w

        for k in range(K):
            pltpu.sync_copy(swb.at[pl.ds(k * CH0, CH0)],
                            sw_hbm.at[_ds8(k * C + cb, CH0)])


@functools.partial(
    pl.kernel,
    out_type=(jax.ShapeDtypeStruct((B * V,), jnp.float32),
              jax.ShapeDtypeStruct((B, L), jnp.float32)),
    mesh=_mesh,
    compiler_params=pltpu.CompilerParams(needs_layout_passes=False),
    scratch_types=[
        pltpu.VMEM((V,), jnp.float32),
        pltpu.VMEM((K * CH,), jnp.int32),
        pltpu.VMEM((K * CH,), jnp.int32),
        pltpu.VMEM((K * CH,), jnp.float32),
        pltpu.VMEM((K * CH,), jnp.float32),
        pltpu.VMEM((ROWS, 128), jnp.int32),
        pltpu.VMEM((ROWS, 128), jnp.float32),
        pltpu.VMEM((L,), jnp.float32),
        pltpu.VMEM((8, L), jnp.float32),
        pltpu.VMEM_SHARED((DSZ,), jnp.float32),
        pltpu.VMEM_SHARED((NS, L), jnp.float32),
        pltpu.SemaphoreType.DMA,
        pltpu.SemaphoreType.DMA,
        pltpu.SemaphoreType.DMA,
    ],
)
def _main(lit_hbm, sw_hbm, a0_hbm, a_out, sat_out,
          a_loc, litb0, litb1, swb0, swb1, idxb, cb, tb, pb,
          delta, parts, sem_in0, sem_in1, sem_sc):
    c = lax.axis_index("c")
    s = lax.axis_index("s")
    b_in = s // 8
    p = lax.rem(s, 8)
    b = 2 * c + b_in
    boff = b_in * V
    tbase = p * TILE_C

    litb = (litb0, litb1)
    swb = (swb0, swb1)
    sem_in = (sem_in0, sem_in1)

    pltpu.sync_copy(a0_hbm.at[_ds8(b * V, V)], a_loc)

    zero16 = jnp.zeros((L,), jnp.float32)
    ones = jnp.ones((L,), jnp.float32)
    iota16 = lax.iota(jnp.int32, L)

    def in_copies(ch, par, n, with_sw):
        cbs = tbase + ch * CH
        cps = []
        for k in range(K):
            cps.append((lit_hbm.at[_ds8(k * C + cbs, n)],
                        litb[par].at[pl.ds(k * n, n)]))
            if with_sw:
                cps.append((sw_hbm.at[_ds8(k * C + cbs, n)],
                            swb[par].at[pl.ds(k * n, n)]))
        return cps

    def start_in(ch, par, n, with_sw):
        for src_, dst in in_copies(ch, par, n, with_sw):
            pltpu.async_copy(src_, dst, sem_in[par])

    def wait_in(ch, par, n, with_sw):
        for src_, dst in in_copies(ch, par, n, with_sw):
            pltpu.make_async_copy(src_, dst, sem_in[par]).wait()

    def fire_scatter(rows):
        for j in range(rows):
            pltpu.async_copy(cb.at[j], delta.at[idxb.at[j]], sem_sc,
                             add=True)

    def drain_scatter(rows):
        for j in range(rows):
            pltpu.make_async_copy(cb.at[j], delta.at[idxb.at[j]],
                                  sem_sc).wait()

    def gather_litvals(par, i, n):
        vars_ = []
        litvals = []
        for k in range(K):
            lit = litb[par][pl.ds(k * n + i * L, L)]
            var = lax.shift_right_logical(lit, 1)
            g = plsc.load_gather(a_loc, [var])
            lv = jnp.where(lax.rem(lit, 2) == 0, g, 1.0 - g)
            vars_.append(var)
            litvals.append(lv)
        return vars_, litvals

    def body(ch, par, n, rows, drain, prefetch):
        wait_in(ch, par, n, True)
        if drain == "always":
            drain_scatter(ROWS)
        elif drain == "cond":
            @pl.when(ch >= 1)
            def _():
                drain_scatter(ROWS)

        @pl.loop(0, n // L)
        def _vec(i):
            vars_, litvals = gather_litvals(par, i, n)
            viol = 1.0 - jnp.maximum(jnp.maximum(litvals[0], litvals[1]),
                                     litvals[2])
            for k in range(K):
                swv = swb[par][pl.ds(k * n + i * L, L)]
                off = k * n + i * L
                r = lax.shift_right_logical(off, 7)
                col = lax.rem(off, 128)
                cb[r, pl.ds(col, L)] = viol * swv
                idxb[r, pl.ds(col, L)] = vars_[k] + boff

        if n == TAIL:
            for j in range(4):
                off = K * TAIL + j * L
                cb[off >> 7, pl.ds(off % 128, L)] = zero16
                idxb[off >> 7, pl.ds(off % 128, L)] = (
                    2 * V + j * L + iota16)
        fire_scatter(rows)
        if prefetch:
            @pl.when(ch + 2 < NCH)
            def _():
                start_in(ch + 2, par, CH, True)

            @pl.when(ch + 2 == NCH)
            def _():
                start_in(NCH, par, TAIL, True)

    @pl.loop(0, ITERS)
    def _iter(it):
        @pl.loop(0, PIECE // L)
        def _z(i):
            swb0[pl.ds(i * L, L)] = zero16

        zb = s * ZSL
        for t in range(5):
            pltpu.sync_copy(swb0.at[pl.ds(0, PIECE)],
                            delta.at[_ds8(zb + t * PIECE, PIECE)])
        pltpu.sync_copy(swb0.at[pl.ds(0, ZSL - 5 * PIECE)],
                        delta.at[_ds8(zb + 5 * PIECE, ZSL - 5 * PIECE)])
        plsc.subcore_barrier()

        with jax.named_scope("scat_pass"):
            start_in(0, 0, CH, True)
            start_in(1, 1, CH, True)

            @pl.loop(0, NCH, step=2)
            def _pair(ch):
                body(ch, 0, CH, ROWS, "cond", True)
                body(ch + 1, 1, CH, ROWS, "always", True)

            body(NCH, 0, TAIL, TROWS, "always", False)
            drain_scatter(TROWS)

        plsc.subcore_barrier()

        def start_piece(j, par, n):
            pltpu.async_copy(delta.at[_ds8(boff + j * PIECE, n)],
                             swb[par].at[pl.ds(0, n)], sem_in[par])

        def apply_piece(j, par, n, pre_j, pre_n):
            pltpu.make_async_copy(delta.at[_ds8(boff + j * PIECE, n)],
                                  swb[par].at[pl.ds(0, n)],
                                  sem_in[par]).wait()
            jb = j * PIECE

            @pl.loop(0, n // L)
            def _v(i):
                av = a_loc[pl.ds(jb + i * L, L)]
                dv = swb[par][pl.ds(i * L, L)]
                a_loc[pl.ds(jb + i * L, L)] = jnp.clip(av + dv, 0.0, 1.0)

            if pre_n:
                start_piece(pre_j, par, pre_n)

        with jax.named_scope("upd_pass"):
            start_piece(0, 0, PIECE)
            start_piece(1, 1, PIECE)

            @pl.loop(0, NPIECE - 3, step=2)
            def _u(j):
                apply_piece(j, 0, PIECE, j + 2, PIECE)
                apply_piece(j + 1, 1, PIECE, j + 3, PIECE)

            apply_piece(NPIECE - 3, 0, PIECE, NPIECE - 1, PIECE)
            apply_piece(NPIECE - 2, 1, PIECE, NPIECE, PTAIL)
            apply_piece(NPIECE - 1, 0, PIECE, 0, 0)
            apply_piece(NPIECE, 1, PTAIL, 0, 0)
        plsc.subcore_barrier()

    @pl.when(lax.rem(p, 2) == 0)
    def _():
        pltpu.sync_copy(a_loc.at[_ds8(p * (V // 8), 2 * (V // 8))],
                        a_out.at[_ds8(b * V + p * (V // 8), 2 * (V // 8))])

    def sat_body(ch, par, n, pv, prefetch):
        wait_in(ch, par, n, False)

        @pl.loop(0, n // L, init_carry=pv)
        def _vec(i, acc):
            _, litvals = gather_litvals(par, i, n)
            csat = jnp.maximum(jnp.maximum(litvals[0], litvals[1]),
                               litvals[2])
            return acc * csat

        if prefetch:
            @pl.when(ch + 2 < NCH)
            def _():
                start_in(ch + 2, par, CH, False)

            @pl.when(ch + 2 == NCH)
            def _():
                start_in(NCH, par, TAIL, False)
        return _vec

    with jax.named_scope("sat_pass"):
        start_in(0, 0, CH, False)
        start_in(1, 1, CH, False)

        @pl.loop(0, NCH, step=2, init_carry=ones)
        def _sat(ch, pv):
            pv = sat_body(ch, 0, CH, pv, True)
            return sat_body(ch + 1, 1, CH, pv, True)

        pv = sat_body(NCH, 0, TAIL, _sat, False)

    for sh in (8, 4, 2, 1):
        tb[...] = pv
        rolled = plsc.load_gather(tb, [lax.rem(iota16 + sh, L)])
        pv = pv * rolled
    tb[...] = pv
    pltpu.sync_copy(tb, parts.at[s])
    plsc.subcore_barrier()

    @pl.when(p == 0)
    def _():
        pltpu.sync_copy(parts.at[pl.ds(b_in * 8, 8)], pb)
        sv = ones
        for k in range(8):
            sv = sv * pb[k]
        tb[...] = sv
        pltpu.sync_copy(tb, sat_out.at[b])


def kernel(clauses, initial_assignment, variable_weights, clause_weights):
    del clause_weights
    lit_t = clauses.T.reshape(-1)
    a0 = initial_assignment.reshape(-1)
    sw = _phase0(lit_t, variable_weights)
    a_flat, sat = _main(lit_t, sw, a0)
    return a_flat.reshape(B, V), sat[:, 0]
